# baseline (speedup 1.0000x reference)
import functools

import jax
import jax.numpy as jnp
from jax import lax
from jax.experimental import pallas as pl
from jax.experimental.pallas import tpu as pltpu
from jax.experimental.pallas import tpu_sc as plsc

SEQ_LEN = 2048
EMBED_DIM = 64
BATCH = 4
TOTAL = BATCH * SEQ_LEN
NUM_WORKERS = 32
ROWS_PER_W = TOTAL // NUM_WORKERS
CHUNK = 128
N_CHUNKS = ROWS_PER_W // CHUNK
LANES = 16


def _body(idx_hbm, tok_hbm, pos_hbm, out_hbm, idx_v, rows_v, pos_v, sem0, sem1):
    wid = lax.axis_index("s") * 2 + lax.axis_index("c")
    base = wid * ROWS_PER_W
    pos_base = lax.rem(base, SEQ_LEN)

    pltpu.sync_copy(idx_hbm.at[pl.ds(wid * N_CHUNKS, N_CHUNKS)], idx_v)
    pltpu.sync_copy(pos_hbm.at[pl.ds(pos_base, ROWS_PER_W)], pos_v)

    cp0 = pltpu.async_copy(tok_hbm.at[idx_v.at[0]],
                           rows_v.at[pl.ds(0, CHUNK)], sem0)
    cp1 = pltpu.async_copy(tok_hbm.at[idx_v.at[1]],
                           rows_v.at[pl.ds(CHUNK, CHUNK)], sem1)

    def add_rows(lo, hi):
        def row_fn(i, carry):
            for c in range(EMBED_DIM // LANES):
                sl = pl.ds(c * LANES, LANES)
                rows_v[i, sl] = rows_v[i, sl] + pos_v[i, sl]
            return carry
        lax.fori_loop(lo, hi, row_fn, 0)

    cp0.wait()
    add_rows(0, CHUNK)
    cp1.wait()
    add_rows(CHUNK, ROWS_PER_W)

    pltpu.sync_copy(rows_v, out_hbm.at[pl.ds(base, ROWS_PER_W)])


@jax.jit
def _run(idx2d, token_table, position_table):
    mesh = plsc.VectorSubcoreMesh(core_axis_name="c", subcore_axis_name="s")
    f = functools.partial(
        pl.kernel,
        out_type=jax.ShapeDtypeStruct((TOTAL, EMBED_DIM), jnp.float32),
        mesh=mesh,
        scratch_types=[
            pltpu.VMEM((N_CHUNKS, CHUNK), jnp.int32),
            pltpu.VMEM((ROWS_PER_W, EMBED_DIM), jnp.float32),
            pltpu.VMEM((ROWS_PER_W, EMBED_DIM), jnp.float32),
            pltpu.SemaphoreType.DMA,
            pltpu.SemaphoreType.DMA,
        ],
        compiler_params=pltpu.CompilerParams(use_tc_tiling_on_sc=False),
    )(_body)
    return f(idx2d, token_table, position_table)


def kernel(inputs, token_table, position_table):
    idx2d = inputs.astype(jnp.int32).reshape(NUM_WORKERS * N_CHUNKS, CHUNK)
    out = _run(idx2d, token_table, position_table)
    return out.reshape(BATCH, SEQ_LEN, EMBED_DIM)


# speedup vs baseline: 1.6399x; 1.6399x over previous
import functools

import jax
import jax.numpy as jnp
from jax import lax
from jax.experimental import pallas as pl
from jax.experimental.pallas import tpu as pltpu
from jax.experimental.pallas import tpu_sc as plsc

SEQ_LEN = 2048
EMBED_DIM = 64
BATCH = 4
TOTAL = BATCH * SEQ_LEN
NUM_WORKERS = 32
ROWS_PER_W = TOTAL // NUM_WORKERS
LANES = 16
N_GROUPS = ROWS_PER_W // LANES


def _body(idx_hbm, tok_hbm, pos_hbm, out_hbm, idx_v, rows_v, pos_v, sem):
    wid = lax.axis_index("s") * 2 + lax.axis_index("c")
    base = wid * ROWS_PER_W
    pos_base = lax.rem(base, SEQ_LEN)

    pltpu.sync_copy(idx_hbm.at[pl.ds(base, ROWS_PER_W)], idx_v)
    pltpu.sync_copy(pos_hbm.at[pl.ds(pos_base, ROWS_PER_W)], pos_v)

    def grp_fn(g, carry):
        j0 = g * LANES
        idx16 = idx_v[pl.ds(j0, LANES)]
        cps = []
        for jj in range(LANES):
            cps.append(pltpu.async_copy(
                tok_hbm.at[idx16[jj]], rows_v.at[j0 + jj], sem))
        for cp in cps:
            cp.wait()
        for jj in range(LANES):
            for c in range(EMBED_DIM // LANES):
                sl = pl.ds(c * LANES, LANES)
                rows_v[j0 + jj, sl] = rows_v[j0 + jj, sl] + pos_v[j0 + jj, sl]
        return carry

    lax.fori_loop(0, N_GROUPS, grp_fn, 0)

    pltpu.sync_copy(rows_v, out_hbm.at[pl.ds(base, ROWS_PER_W)])


@jax.jit
def _run(idx, token_table, position_table):
    mesh = plsc.VectorSubcoreMesh(core_axis_name="c", subcore_axis_name="s")
    f = functools.partial(
        pl.kernel,
        out_type=jax.ShapeDtypeStruct((TOTAL, EMBED_DIM), jnp.float32),
        mesh=mesh,
        scratch_types=[
            pltpu.VMEM((ROWS_PER_W,), jnp.int32),
            pltpu.VMEM((ROWS_PER_W, EMBED_DIM), jnp.float32),
            pltpu.VMEM((ROWS_PER_W, EMBED_DIM), jnp.float32),
            pltpu.SemaphoreType.DMA,
        ],
    )(_body)
    return f(idx, token_table, position_table)


def kernel(inputs, token_table, position_table):
    idx = inputs.astype(jnp.int32).reshape(TOTAL)
    out = _run(idx, token_table, position_table)
    return out.reshape(BATCH, SEQ_LEN, EMBED_DIM)


# speedup vs baseline: 1.6760x; 1.0220x over previous
import functools

import jax
import jax.numpy as jnp
from jax import lax
from jax.experimental import pallas as pl
from jax.experimental.pallas import tpu as pltpu
from jax.experimental.pallas import tpu_sc as plsc

SEQ_LEN = 2048
EMBED_DIM = 64
BATCH = 4
TOTAL = BATCH * SEQ_LEN
NUM_WORKERS = 32
ROWS_PER_W = TOTAL // NUM_WORKERS
LANES = 16
N_GROUPS = ROWS_PER_W // LANES


def _body(idx_hbm, tok_hbm, pos_hbm, out_hbm, idx_v, rows_v, pos_v, *sems):
    wid = lax.axis_index("s") * 2 + lax.axis_index("c")
    base = wid * ROWS_PER_W
    pos_base = lax.rem(base, SEQ_LEN)

    pltpu.sync_copy(idx_hbm.at[pl.ds(base, ROWS_PER_W)], idx_v)
    pltpu.sync_copy(pos_hbm.at[pl.ds(pos_base, ROWS_PER_W)], pos_v)

    for g in range(N_GROUPS):
        j0 = g * LANES
        idx16 = idx_v[pl.ds(j0, LANES)]
        for jj in range(LANES):
            pltpu.async_copy(tok_hbm.at[idx16[jj]], rows_v.at[j0 + jj], sems[g])

    for g in range(N_GROUPS):
        pltpu.make_async_copy(
            tok_hbm.at[pl.ds(0, LANES)],
            rows_v.at[pl.ds(g * LANES, LANES)], sems[g]
        ).wait()

    def grp_fn(g, carry):
        j0 = g * LANES
        for jj in range(LANES):
            for c in range(EMBED_DIM // LANES):
                sl = pl.ds(c * LANES, LANES)
                rows_v[j0 + jj, sl] = rows_v[j0 + jj, sl] + pos_v[j0 + jj, sl]
        return carry

    lax.fori_loop(0, N_GROUPS, grp_fn, 0)

    pltpu.sync_copy(rows_v, out_hbm.at[pl.ds(base, ROWS_PER_W)])


@jax.jit
def _run(idx, token_table, position_table):
    mesh = plsc.VectorSubcoreMesh(core_axis_name="c", subcore_axis_name="s")
    f = functools.partial(
        pl.kernel,
        out_type=jax.ShapeDtypeStruct((TOTAL, EMBED_DIM), jnp.float32),
        mesh=mesh,
        scratch_types=[
            pltpu.VMEM((ROWS_PER_W,), jnp.int32),
            pltpu.VMEM((ROWS_PER_W, EMBED_DIM), jnp.float32),
            pltpu.VMEM((ROWS_PER_W, EMBED_DIM), jnp.float32),
        ] + [pltpu.SemaphoreType.DMA] * N_GROUPS,
    )(_body)
    return f(idx, token_table, position_table)


def kernel(inputs, token_table, position_table):
    idx = inputs.astype(jnp.int32).reshape(TOTAL)
    out = _run(idx, token_table, position_table)
    return out.reshape(BATCH, SEQ_LEN, EMBED_DIM)
